# baseline (speedup 1.0000x reference)
import functools

import jax
import jax.numpy as jnp
import numpy as np
from jax import lax
from jax.experimental import pallas as pl
from jax.experimental.pallas import tpu as pltpu
from jax.experimental.pallas import tpu_sc as plsc

Qn, Nn, Pn = 32, 100000, 128
L = 16
NPAD = 100352
BNP = NPAD // 4
NBP = 4
NSEG = NPAD // Pn
NROWS = NSEG * Qn
QTILE = BNP // Pn
NCHUNK0 = NPAD // 256
L1PAD = 25 * 256
L2PAD = 2 * 256
KMAX = 64
GCH = 112
BIG = np.float32(3.0e38)


def _dist_kernel(q_ref, keys_ref, out_ref):
    j = pl.program_id(0)
    q = q_ref[...]
    qn = q / jnp.sum(q, axis=1, keepdims=True)
    q2 = jnp.sum(qn * qn, axis=1, keepdims=True)
    m = keys_ref[...]
    aug = jnp.concatenate([qn, jnp.ones((8, Pn), jnp.float32)], axis=0)
    g = lax.dot_general(aug, m, (((1,), (1,)), ((), ())),
                        preferred_element_type=jnp.float32)
    qm = g[:Qn]
    s = g[Qn:Qn + 1]
    t2 = lax.dot_general(jnp.ones((8, Pn), jnp.float32), m * m,
                         (((1,), (1,)), ((), ())),
                         preferred_element_type=jnp.float32)[0:1]
    inv = 1.0 / s
    d2 = q2 - 2.0 * qm * inv + t2 * inv * inv
    dist = jnp.sqrt(jnp.maximum(d2, 0.0))
    col = jax.lax.broadcasted_iota(jnp.int32, (Qn, BNP), 1) + j * BNP
    dist = jnp.where(col >= Nn, BIG, dist)
    v = dist.reshape(4, 8, QTILE, Pn).transpose(0, 2, 1, 3)
    out_ref[...] = v.reshape(NROWS // NBP, Pn)


def _dist(queries, keys):
    return pl.pallas_call(
        _dist_kernel,
        grid=(NBP,),
        in_specs=[
            pl.BlockSpec((Qn, Pn), lambda i: (0, 0)),
            pl.BlockSpec((BNP, Pn), lambda i: (i, 0)),
        ],
        out_specs=pl.BlockSpec((NROWS // NBP, Pn), lambda i: (i, 0)),
        out_shape=jax.ShapeDtypeStruct((NROWS, Pn), jnp.float32),
    )(queries, keys)


def _ffs(mask):
    r = plsc.all_reduce_ffs(mask)
    return r[0] if r.ndim else r


def _sc_body(dist_hbm, ctr_hbm, kk_hbm, uc_hbm, ua_hbm, out_hbm,
             raw, l1, l2, l3, gidx, nbr_idx, nbr_w, rows, ucv, uav, scorev,
             kv, sem, sems):
    nc = 2
    wid = lax.axis_index("s") * nc + lax.axis_index("c")
    q = wid
    qb = q // 8
    qs = q % 8
    iota = lax.iota(jnp.int32, L)
    iotaf = iota.astype(jnp.float32)
    mask0 = iota == 0
    zero16i = iota * 0
    zero16f = iotaf * 0.0
    inf16 = zero16f + BIG

    def _sload(ref, pos):
        return plsc.load_gather(ref, [zero16i + pos])[0]

    def _sstore(ref, pos, val):
        plsc.store_scatter(ref, [zero16i + pos], zero16f * 0 + val
                           if jnp.asarray(val).dtype == jnp.float32
                           else zero16i + val, mask=mask0)

    def _sstore2(ref, r, c, val):
        plsc.store_scatter(ref, [zero16i + r, zero16i + c],
                           zero16f * 0 + val, mask=mask0)

    for v in range(NSEG // L):
        i = iota + v * L
        jj = i // QTILE
        lt = i % QTILE
        gidx[pl.ds(v * L, L)] = (jj * 4 + qb) * QTILE * 8 + lt * 8 + qs
    pltpu.sync_copy(kk_hbm, kv)
    for u in range(KMAX // L):
        nbr_w[pl.ds(u * L, L)] = inf16
        nbr_idx[pl.ds(u * L, L)] = zero16i

    cps = [pltpu.async_copy(dist_hbm.at[gidx.at[pl.ds(c * GCH, GCH)]],
                            raw.at[pl.ds(c * GCH, GCH)], sems[c])
           for c in range(NSEG // GCH)]
    pltpu.sync_copy(uc_hbm.at[q], ucv)
    pltpu.sync_copy(ua_hbm.at[q], uav)
    kk = kv[pl.ds(0, L)][0]

    def build(src, dst, lo, hi, unroll=4):
        @plsc.parallel_loop(lo, hi, unroll=unroll)
        def _(c):
            acc = src[2 * c, pl.ds(0, L)]
            for u in range(1, 16):
                acc = jnp.minimum(acc, src[2 * c + u // 8,
                                           pl.ds((u % 8) * L, L)])
            dst[pl.ds(c * L, L)] = acc

    def build1(src, dst, nchunk, unroll=4):
        @plsc.parallel_loop(0, nchunk, unroll=unroll)
        def _(c):
            base = c * 256
            acc = src[pl.ds(base, L)]
            for u in range(1, 16):
                acc = jnp.minimum(acc, src[pl.ds(base + u * L, L)])
            dst[pl.ds(c * L, L)] = acc

    ch_per = GCH * Pn // 256
    with jax.named_scope("sc_build"):
        for c, cp in enumerate(cps):
            cp.wait()
            build(raw, l1, c * ch_per, (c + 1) * ch_per)
    for u in range((L1PAD - NCHUNK0 * L) // L):
        l1[pl.ds(NCHUNK0 * L + u * L, L)] = inf16
    build1(l1, l2, L1PAD // 256)
    for u in range((L2PAD - (L1PAD // 256) * L) // L):
        l2[pl.ds((L1PAD // 256) * L + u * L, L)] = inf16
    build1(l2, l3, 2)

    def extract(i, _t):
        v0 = l3[pl.ds(0, L)]
        v1 = l3[pl.ds(L, L)]
        m0 = jnp.min(v0)
        m1 = jnp.min(v1)
        m = jnp.minimum(m0, m1)
        first = m0 <= m1
        c3 = jnp.where(first, 0, 1).astype(jnp.int32)
        vsel = jnp.where(first, v0, v1)
        lane3 = _ffs(vsel == m)
        base2 = c3 * 256 + lane3
        cand2 = plsc.load_gather(l2, [base2 + iota * L])
        u2 = _ffs(cand2 == m)
        f = base2 + u2 * L
        base1 = (f // L) * 256 + (f % L)
        cand1 = plsc.load_gather(l1, [base1 + iota * L])
        u1 = _ffs(cand1 == m)
        e = base1 + u1 * L
        base0 = (e // L) * 256 + (e % L)
        flat0 = base0 + iota * L
        cand0 = plsc.load_gather(raw, [flat0 // Pn, flat0 % Pn])
        u0 = _ffs(cand0 == m)
        r = base0 + u0 * L
        _sstore(nbr_idx, i, r)
        _sstore(nbr_w, i, m)
        _sstore2(raw, r // Pn, r % Pn, m * 0.0 + BIG)
        nm1 = jnp.min(jnp.where(iota == u0, BIG, cand0))
        _sstore(l1, e, nm1)
        nm2 = jnp.min(jnp.where(iota == u1, nm1, cand1))
        _sstore(l2, f, nm2)
        nm3 = jnp.min(jnp.where(iota == u2, nm2, cand2))
        _sstore(l3, c3 * L + lane3, nm3)
        return m

    with jax.named_scope("sc_extract"):
        t = lax.fori_loop(0, kk, extract, np.float32(0.0))

    for u in range(KMAX // L):
        dv = nbr_w[pl.ds(u * L, L)]
        nbr_w[pl.ds(u * L, L)] = jnp.maximum(t - dv, 0.0)

    pltpu.async_copy(ctr_hbm.at[nbr_idx], rows, sem).wait()

    def accum(i, acc):
        w = _sload(nbr_w, i)
        return tuple(acc[u] + w * rows[i, pl.ds(u * L, L)] for u in range(8))

    with jax.named_scope("sc_accum"):
        acc = lax.fori_loop(0, KMAX, accum,
                            tuple(zero16f for _ in range(8)))

    s = np.float32(0.0)
    for u in range(8):
        av = uav[pl.ds(u * L, L)]
        cv = ucv[pl.ds(u * L, L)]
        denom = jnp.where(av != 0.0, av, 1.0)
        ratio = jnp.where(av != 0.0, cv / denom, 0.0)
        sv = acc[u] + t * ratio
        scorev[pl.ds(u * L, L)] = sv
        s = s + jnp.sum(sv)
    for u in range(8):
        scorev[pl.ds(u * L, L)] = scorev[pl.ds(u * L, L)] / s
    pltpu.sync_copy(scorev, out_hbm.at[q])


_sc_score = functools.partial(
    pl.kernel,
    out_type=jax.ShapeDtypeStruct((Qn, Pn), jnp.float32),
    mesh=plsc.VectorSubcoreMesh(core_axis_name="c", subcore_axis_name="s"),
    compiler_params=pltpu.CompilerParams(needs_layout_passes=False,
                                         use_tc_tiling_on_sc=False),
    scratch_types=[
        pltpu.VMEM((NSEG, Pn), jnp.float32),
        pltpu.VMEM((L1PAD,), jnp.float32),
        pltpu.VMEM((L2PAD,), jnp.float32),
        pltpu.VMEM((2 * L,), jnp.float32),
        pltpu.VMEM((NSEG,), jnp.int32),
        pltpu.VMEM((KMAX,), jnp.int32),
        pltpu.VMEM((KMAX,), jnp.float32),
        pltpu.VMEM((KMAX, Pn), jnp.float32),
        pltpu.VMEM((Pn,), jnp.float32),
        pltpu.VMEM((Pn,), jnp.float32),
        pltpu.VMEM((Pn,), jnp.float32),
        pltpu.VMEM((L,), jnp.int32),
        pltpu.SemaphoreType.DMA,
        [pltpu.SemaphoreType.DMA] * (NSEG // GCH),
    ],
)(_sc_body)


def kernel(queries, keys, ctr, user_clicks, user_attempts, k):
    dist = _dist(queries, keys)
    kk = jnp.full((L,), jnp.asarray(k, jnp.int32))
    return _sc_score(dist, ctr, kk, user_clicks, user_attempts)


# speedup vs baseline: 1.2097x; 1.2097x over previous
import functools

import jax
import jax.numpy as jnp
import numpy as np
from jax import lax
from jax.experimental import pallas as pl
from jax.experimental.pallas import tpu as pltpu
from jax.experimental.pallas import tpu_sc as plsc

Qn, Nn, Pn = 32, 100000, 128
L = 16
NPAD = 100352
BNP = NPAD // 4
NBP = 4
NSEG = NPAD // Pn
NROWS = NSEG * Qn
QTILE = BNP // Pn
NCHUNK0 = NPAD // 256
L1PAD = 25 * 256
L2PAD = 2 * 256
KMAX = 64
GCH = 112
BIG = np.float32(3.0e38)


def _dist_kernel(q_ref, keys_ref, out_ref):
    j = pl.program_id(0)
    q = q_ref[...]
    qn = q / jnp.sum(q, axis=1, keepdims=True)
    q2 = jnp.sum(qn * qn, axis=1, keepdims=True)
    m = keys_ref[...]
    aug = jnp.concatenate([qn, jnp.ones((8, Pn), jnp.float32)], axis=0)
    g = lax.dot_general(aug, m, (((1,), (1,)), ((), ())),
                        preferred_element_type=jnp.float32)
    qm = g[:Qn]
    s = g[Qn:Qn + 1]
    t2 = lax.dot_general(jnp.ones((8, Pn), jnp.float32), m * m,
                         (((1,), (1,)), ((), ())),
                         preferred_element_type=jnp.float32)[0:1]
    inv = 1.0 / s
    d2 = q2 - 2.0 * qm * inv + t2 * inv * inv
    dist = jnp.sqrt(jnp.maximum(d2, 0.0))
    col = jax.lax.broadcasted_iota(jnp.int32, (Qn, BNP), 1) + j * BNP
    dist = jnp.where(col >= Nn, BIG, dist)
    out_ref[...] = dist.reshape(NROWS // NBP, Pn)


def _dist(queries, keys):
    return pl.pallas_call(
        _dist_kernel,
        grid=(NBP,),
        in_specs=[
            pl.BlockSpec((Qn, Pn), lambda i: (0, 0)),
            pl.BlockSpec((BNP, Pn), lambda i: (i, 0)),
        ],
        out_specs=pl.BlockSpec((NROWS // NBP, Pn), lambda i: (i, 0)),
        out_shape=jax.ShapeDtypeStruct((NROWS, Pn), jnp.float32),
    )(queries, keys)


def _ffs(mask):
    r = plsc.all_reduce_ffs(mask)
    return r[0] if r.ndim else r


def _sc_body(dist_hbm, ctr_hbm, kk_hbm, uc_hbm, ua_hbm, out_hbm,
             raw, l1, l2, l3, nbr_idx, nbr_w, rows, ucv, uav, scorev,
             kv, sem, sems):
    nc = 2
    wid = lax.axis_index("s") * nc + lax.axis_index("c")
    q = wid
    qb = q // 8
    qs = q % 8
    iota = lax.iota(jnp.int32, L)
    iotaf = iota.astype(jnp.float32)
    mask0 = iota == 0
    zero16i = iota * 0
    zero16f = iotaf * 0.0
    inf16 = zero16f + BIG

    def _sload(ref, pos):
        return plsc.load_gather(ref, [zero16i + pos])[0]

    def _sstore(ref, pos, val):
        plsc.store_scatter(ref, [zero16i + pos], zero16f * 0 + val
                           if jnp.asarray(val).dtype == jnp.float32
                           else zero16i + val, mask=mask0)

    def _sstore2(ref, r, c, val):
        plsc.store_scatter(ref, [zero16i + r, zero16i + c],
                           zero16f * 0 + val, mask=mask0)

    pltpu.sync_copy(kk_hbm, kv)
    for u in range(KMAX // L):
        nbr_w[pl.ds(u * L, L)] = inf16
        nbr_idx[pl.ds(u * L, L)] = zero16i

    cps = [pltpu.async_copy(
               dist_hbm.at[pl.ds(j * (NROWS // NBP) + q * QTILE, QTILE)],
               raw.at[pl.ds(j * QTILE, QTILE)], sems[j])
           for j in range(NBP)]
    pltpu.sync_copy(uc_hbm.at[q], ucv)
    pltpu.sync_copy(ua_hbm.at[q], uav)
    kk = kv[pl.ds(0, L)][0]

    def build(src, dst, lo, hi, unroll=4):
        @plsc.parallel_loop(lo, hi, unroll=unroll)
        def _(c):
            acc = src[2 * c, pl.ds(0, L)]
            for u in range(1, 16):
                acc = jnp.minimum(acc, src[2 * c + u // 8,
                                           pl.ds((u % 8) * L, L)])
            dst[pl.ds(c * L, L)] = acc

    def build1(src, dst, nchunk, unroll=4):
        @plsc.parallel_loop(0, nchunk, unroll=unroll)
        def _(c):
            base = c * 256
            acc = src[pl.ds(base, L)]
            for u in range(1, 16):
                acc = jnp.minimum(acc, src[pl.ds(base + u * L, L)])
            dst[pl.ds(c * L, L)] = acc

    ch_per = BNP // 256
    with jax.named_scope("sc_build"):
        for c, cp in enumerate(cps):
            cp.wait()
            build(raw, l1, c * ch_per, (c + 1) * ch_per)
    for u in range((L1PAD - NCHUNK0 * L) // L):
        l1[pl.ds(NCHUNK0 * L + u * L, L)] = inf16
    build1(l1, l2, L1PAD // 256)
    for u in range((L2PAD - (L1PAD // 256) * L) // L):
        l2[pl.ds((L1PAD // 256) * L + u * L, L)] = inf16
    build1(l2, l3, 2)

    def extract(i, _t):
        v0 = l3[pl.ds(0, L)]
        v1 = l3[pl.ds(L, L)]
        m0 = jnp.min(v0)
        m1 = jnp.min(v1)
        m = jnp.minimum(m0, m1)
        first = m0 <= m1
        c3 = jnp.where(first, 0, 1).astype(jnp.int32)
        vsel = jnp.where(first, v0, v1)
        lane3 = _ffs(vsel == m)
        base2 = c3 * 256 + lane3
        cand2 = plsc.load_gather(l2, [base2 + iota * L])
        u2 = _ffs(cand2 == m)
        f = base2 + u2 * L
        base1 = (f // L) * 256 + (f % L)
        cand1 = plsc.load_gather(l1, [base1 + iota * L])
        u1 = _ffs(cand1 == m)
        e = base1 + u1 * L
        base0 = (e // L) * 256 + (e % L)
        flat0 = base0 + iota * L
        cand0 = plsc.load_gather(raw, [flat0 // Pn, flat0 % Pn])
        u0 = _ffs(cand0 == m)
        r = base0 + u0 * L
        _sstore(nbr_idx, i, r)
        _sstore(nbr_w, i, m)
        pltpu.async_copy(ctr_hbm.at[r], rows.at[i], sem)
        _sstore2(raw, r // Pn, r % Pn, m * 0.0 + BIG)
        nm1 = jnp.min(jnp.where(iota == u0, BIG, cand0))
        _sstore(l1, e, nm1)
        nm2 = jnp.min(jnp.where(iota == u1, nm1, cand1))
        _sstore(l2, f, nm2)
        nm3 = jnp.min(jnp.where(iota == u2, nm2, cand2))
        _sstore(l3, c3 * L + lane3, nm3)
        return m

    with jax.named_scope("sc_extract"):
        t = lax.fori_loop(0, kk, extract, np.float32(0.0))

    for u in range(KMAX // L):
        dv = nbr_w[pl.ds(u * L, L)]
        nbr_w[pl.ds(u * L, L)] = jnp.maximum(t - dv, 0.0)

    def drain(i, z):
        pltpu.make_async_copy(ctr_hbm.at[0], rows.at[0], sem).wait()
        return z
    lax.fori_loop(0, kk, drain, 0)

    def accum(i, acc):
        w = _sload(nbr_w, i)
        return tuple(acc[u] + w * rows[i, pl.ds(u * L, L)] for u in range(8))

    with jax.named_scope("sc_accum"):
        acc = lax.fori_loop(0, kk, accum,
                            tuple(zero16f for _ in range(8)))

    s = np.float32(0.0)
    for u in range(8):
        av = uav[pl.ds(u * L, L)]
        cv = ucv[pl.ds(u * L, L)]
        denom = jnp.where(av != 0.0, av, 1.0)
        ratio = jnp.where(av != 0.0, cv / denom, 0.0)
        sv = acc[u] + t * ratio
        scorev[pl.ds(u * L, L)] = sv
        s = s + jnp.sum(sv)
    for u in range(8):
        scorev[pl.ds(u * L, L)] = scorev[pl.ds(u * L, L)] / s
    pltpu.sync_copy(scorev, out_hbm.at[q])


_sc_score = functools.partial(
    pl.kernel,
    out_type=jax.ShapeDtypeStruct((Qn, Pn), jnp.float32),
    mesh=plsc.VectorSubcoreMesh(core_axis_name="c", subcore_axis_name="s"),
    compiler_params=pltpu.CompilerParams(needs_layout_passes=False,
                                         use_tc_tiling_on_sc=False),
    scratch_types=[
        pltpu.VMEM((NSEG, Pn), jnp.float32),
        pltpu.VMEM((L1PAD,), jnp.float32),
        pltpu.VMEM((L2PAD,), jnp.float32),
        pltpu.VMEM((2 * L,), jnp.float32),
        pltpu.VMEM((KMAX,), jnp.int32),
        pltpu.VMEM((KMAX,), jnp.float32),
        pltpu.VMEM((KMAX, Pn), jnp.float32),
        pltpu.VMEM((Pn,), jnp.float32),
        pltpu.VMEM((Pn,), jnp.float32),
        pltpu.VMEM((Pn,), jnp.float32),
        pltpu.VMEM((L,), jnp.int32),
        pltpu.SemaphoreType.DMA,
        [pltpu.SemaphoreType.DMA] * NBP,
    ],
)(_sc_body)


def kernel(queries, keys, ctr, user_clicks, user_attempts, k):
    dist = _dist(queries, keys)
    kk = jnp.full((L,), jnp.asarray(k, jnp.int32))
    return _sc_score(dist, ctr, kk, user_clicks, user_attempts)


# speedup vs baseline: 1.2197x; 1.0083x over previous
import functools

import jax
import jax.numpy as jnp
import numpy as np
from jax import lax
from jax.experimental import pallas as pl
from jax.experimental.pallas import tpu as pltpu
from jax.experimental.pallas import tpu_sc as plsc

Qn, Nn, Pn = 32, 100000, 128
L = 16
NPAD = 100352
BNP = NPAD // 8
NBP = 8
NSEG = NPAD // Pn
NROWS = NSEG * Qn
QTILE = BNP // Pn
NCHUNK0 = NPAD // 256
L1PAD = 25 * 256
L2PAD = 2 * 256
KMAX = 64
GCH = 112
BIG = np.float32(3.0e38)


def _dist_kernel(q_ref, keys_ref, out_ref):
    j = pl.program_id(0)
    q = q_ref[...]
    qn = q / jnp.sum(q, axis=1, keepdims=True)
    q2 = jnp.sum(qn * qn, axis=1, keepdims=True)
    m = keys_ref[...]
    aug = jnp.concatenate([qn, jnp.ones((8, Pn), jnp.float32)], axis=0)
    g = lax.dot_general(aug, m, (((1,), (1,)), ((), ())),
                        preferred_element_type=jnp.float32)
    qm = g[:Qn]
    s = g[Qn:Qn + 1]
    t2 = lax.dot_general(jnp.ones((8, Pn), jnp.float32), m * m,
                         (((1,), (1,)), ((), ())),
                         preferred_element_type=jnp.float32)[0:1]
    inv = 1.0 / s
    d2 = q2 - 2.0 * qm * inv + t2 * inv * inv
    dist = jnp.sqrt(jnp.maximum(d2, 0.0))
    col = jax.lax.broadcasted_iota(jnp.int32, (Qn, BNP), 1) + j * BNP
    dist = jnp.where(col >= Nn, BIG, dist)
    out_ref[...] = dist.reshape(NROWS // NBP, Pn)


def _dist(queries, keys):
    return pl.pallas_call(
        _dist_kernel,
        grid=(NBP,),
        in_specs=[
            pl.BlockSpec((Qn, Pn), lambda i: (0, 0)),
            pl.BlockSpec((BNP, Pn), lambda i: (i, 0)),
        ],
        out_specs=pl.BlockSpec((NROWS // NBP, Pn), lambda i: (i, 0)),
        out_shape=jax.ShapeDtypeStruct((NROWS, Pn), jnp.float32),
    )(queries, keys)


def _ffs(mask):
    r = plsc.all_reduce_ffs(mask)
    return r[0] if r.ndim else r


def _sc_body(dist_hbm, ctr_hbm, kk_hbm, uc_hbm, ua_hbm, out_hbm,
             raw, l1, l2, l3, nbr_idx, nbr_w, rows, ucv, uav, scorev,
             kv, sem, sems):
    nc = 2
    wid = lax.axis_index("s") * nc + lax.axis_index("c")
    q = wid
    qb = q // 8
    qs = q % 8
    iota = lax.iota(jnp.int32, L)
    iotaf = iota.astype(jnp.float32)
    mask0 = iota == 0
    zero16i = iota * 0
    zero16f = iotaf * 0.0
    inf16 = zero16f + BIG

    def _sload(ref, pos):
        return plsc.load_gather(ref, [zero16i + pos])[0]

    def _sstore(ref, pos, val):
        plsc.store_scatter(ref, [zero16i + pos], zero16f * 0 + val
                           if jnp.asarray(val).dtype == jnp.float32
                           else zero16i + val, mask=mask0)

    def _sstore2(ref, r, c, val):
        plsc.store_scatter(ref, [zero16i + r, zero16i + c],
                           zero16f * 0 + val, mask=mask0)

    pltpu.sync_copy(kk_hbm, kv)
    for u in range(KMAX // L):
        nbr_w[pl.ds(u * L, L)] = inf16
        nbr_idx[pl.ds(u * L, L)] = zero16i

    cps = [pltpu.async_copy(
               dist_hbm.at[pl.ds(j * (NROWS // NBP) + q * QTILE, QTILE)],
               raw.at[pl.ds(j * QTILE, QTILE)], sems[j])
           for j in range(NBP)]
    pltpu.sync_copy(uc_hbm.at[q], ucv)
    pltpu.sync_copy(ua_hbm.at[q], uav)
    kk = kv[pl.ds(0, L)][0]

    def build(src, dst, lo, hi, unroll=4):
        @plsc.parallel_loop(lo, hi, unroll=unroll)
        def _(c):
            acc = src[2 * c, pl.ds(0, L)]
            for u in range(1, 16):
                acc = jnp.minimum(acc, src[2 * c + u // 8,
                                           pl.ds((u % 8) * L, L)])
            dst[pl.ds(c * L, L)] = acc

    def build1(src, dst, nchunk, unroll=4):
        @plsc.parallel_loop(0, nchunk, unroll=unroll)
        def _(c):
            base = c * 256
            acc = src[pl.ds(base, L)]
            for u in range(1, 16):
                acc = jnp.minimum(acc, src[pl.ds(base + u * L, L)])
            dst[pl.ds(c * L, L)] = acc

    ch_per = BNP // 256
    with jax.named_scope("sc_build"):
        for c, cp in enumerate(cps):
            cp.wait()
            build(raw, l1, c * ch_per, (c + 1) * ch_per)
    for u in range((L1PAD - NCHUNK0 * L) // L):
        l1[pl.ds(NCHUNK0 * L + u * L, L)] = inf16
    build1(l1, l2, L1PAD // 256)
    for u in range((L2PAD - (L1PAD // 256) * L) // L):
        l2[pl.ds((L1PAD // 256) * L + u * L, L)] = inf16
    build1(l2, l3, 2)

    def extract(i, _t):
        v0 = l3[pl.ds(0, L)]
        v1 = l3[pl.ds(L, L)]
        m0 = jnp.min(v0)
        m1 = jnp.min(v1)
        m = jnp.minimum(m0, m1)
        first = m0 <= m1
        c3 = jnp.where(first, 0, 1).astype(jnp.int32)
        vsel = jnp.where(first, v0, v1)
        lane3 = _ffs(vsel == m)
        base2 = c3 * 256 + lane3
        cand2 = plsc.load_gather(l2, [base2 + iota * L])
        u2 = _ffs(cand2 == m)
        f = base2 + u2 * L
        base1 = (f // L) * 256 + (f % L)
        cand1 = plsc.load_gather(l1, [base1 + iota * L])
        u1 = _ffs(cand1 == m)
        e = base1 + u1 * L
        base0 = (e // L) * 256 + (e % L)
        flat0 = base0 + iota * L
        cand0 = plsc.load_gather(raw, [flat0 // Pn, flat0 % Pn])
        u0 = _ffs(cand0 == m)
        r = base0 + u0 * L
        _sstore(nbr_idx, i, r)
        _sstore(nbr_w, i, m)
        pltpu.async_copy(ctr_hbm.at[r], rows.at[i], sem)
        _sstore2(raw, r // Pn, r % Pn, m * 0.0 + BIG)
        nm1 = jnp.min(jnp.where(iota == u0, BIG, cand0))
        _sstore(l1, e, nm1)
        nm2 = jnp.min(jnp.where(iota == u1, nm1, cand1))
        _sstore(l2, f, nm2)
        nm3 = jnp.min(jnp.where(iota == u2, nm2, cand2))
        _sstore(l3, c3 * L + lane3, nm3)
        return m

    with jax.named_scope("sc_extract"):
        t = lax.fori_loop(0, kk, extract, np.float32(0.0))

    for u in range(KMAX // L):
        dv = nbr_w[pl.ds(u * L, L)]
        nbr_w[pl.ds(u * L, L)] = jnp.maximum(t - dv, 0.0)

    def drain(i, z):
        pltpu.make_async_copy(ctr_hbm.at[0], rows.at[0], sem).wait()
        return z
    lax.fori_loop(0, kk, drain, 0)

    def accum(i, acc):
        w = _sload(nbr_w, i)
        return tuple(acc[u] + w * rows[i, pl.ds(u * L, L)] for u in range(8))

    with jax.named_scope("sc_accum"):
        acc = lax.fori_loop(0, kk, accum,
                            tuple(zero16f for _ in range(8)))

    s = np.float32(0.0)
    for u in range(8):
        av = uav[pl.ds(u * L, L)]
        cv = ucv[pl.ds(u * L, L)]
        denom = jnp.where(av != 0.0, av, 1.0)
        ratio = jnp.where(av != 0.0, cv / denom, 0.0)
        sv = acc[u] + t * ratio
        scorev[pl.ds(u * L, L)] = sv
        s = s + jnp.sum(sv)
    for u in range(8):
        scorev[pl.ds(u * L, L)] = scorev[pl.ds(u * L, L)] / s
    pltpu.sync_copy(scorev, out_hbm.at[q])


_sc_score = functools.partial(
    pl.kernel,
    out_type=jax.ShapeDtypeStruct((Qn, Pn), jnp.float32),
    mesh=plsc.VectorSubcoreMesh(core_axis_name="c", subcore_axis_name="s"),
    compiler_params=pltpu.CompilerParams(needs_layout_passes=False,
                                         use_tc_tiling_on_sc=False),
    scratch_types=[
        pltpu.VMEM((NSEG, Pn), jnp.float32),
        pltpu.VMEM((L1PAD,), jnp.float32),
        pltpu.VMEM((L2PAD,), jnp.float32),
        pltpu.VMEM((2 * L,), jnp.float32),
        pltpu.VMEM((KMAX,), jnp.int32),
        pltpu.VMEM((KMAX,), jnp.float32),
        pltpu.VMEM((KMAX, Pn), jnp.float32),
        pltpu.VMEM((Pn,), jnp.float32),
        pltpu.VMEM((Pn,), jnp.float32),
        pltpu.VMEM((Pn,), jnp.float32),
        pltpu.VMEM((L,), jnp.int32),
        pltpu.SemaphoreType.DMA,
        [pltpu.SemaphoreType.DMA] * NBP,
    ],
)(_sc_body)


def kernel(queries, keys, ctr, user_clicks, user_attempts, k):
    dist = _dist(queries, keys)
    kk = jnp.full((L,), jnp.asarray(k, jnp.int32))
    return _sc_score(dist, ctr, kk, user_clicks, user_attempts)
